# baseline (speedup 1.0000x reference)
import functools

import jax
import jax.numpy as jnp
from jax import lax
from jax.experimental import pallas as pl
from jax.experimental.pallas import tpu as pltpu
from jax.experimental.pallas import tpu_sc as plsc

F32 = jnp.float32

H = 128
NA = 50001
NB = 200001
NMOL = 1000
MOLSZ = 50
MAXB = 6
NA_PAD = 53248
NB_PAD = 200704

NC = 2
NS = 16
NW = NC * NS

ATOMS_W = NA_PAD // NW
A_BLK = 64
N_ABLK = ATOMS_W // A_BLK
BONDS_W = NB_PAD // NW
B_BLK = 112
N_BBLK = BONDS_W // B_BLK

ROW_BLK = 2048


def _sc_mesh():
    return plsc.VectorSubcoreMesh(core_axis_name="c", subcore_axis_name="s")


def _sc_gather6(mb, idx_sm):
    G_STRM = 104
    G_BLK = 2 * G_STRM
    GW = MAXB * NA_PAD // NW
    N_GBLK = GW // G_BLK

    scratch = []
    for _ in range(2):
        scratch += [
            pltpu.VMEM((G_STRM,), jnp.int32),
            pltpu.VMEM((G_STRM,), jnp.int32),
            pltpu.VMEM((G_STRM, H), F32),
            pltpu.VMEM((G_STRM, H), F32),
            pltpu.SemaphoreType.DMA,
            pltpu.SemaphoreType.DMA,
        ]

    def body(mb_hbm, idx_hbm, out_hbm, *bufs):
        idx0_v = (bufs[0], bufs[6])
        idx1_v = (bufs[1], bufs[7])
        row0_v = (bufs[2], bufs[8])
        row1_v = (bufs[3], bufs[9])
        gsem = (bufs[4], bufs[10])
        osem = (bufs[5], bufs[11])
        wid = lax.axis_index("s") * NC + lax.axis_index("c")
        e_base = wid * GW

        def gathers(b):
            return [
                pltpu.make_async_copy(mb_hbm.at[idx0_v[b]], row0_v[b],
                                      gsem[b]),
                pltpu.make_async_copy(mb_hbm.at[idx1_v[b]], row1_v[b],
                                      gsem[b]),
            ]

        def issue_block(k, b):
            e0 = e_base + k * G_BLK
            pltpu.sync_copy(idx_hbm.at[pl.ds(e0, G_STRM)], idx0_v[b])
            pltpu.sync_copy(idx_hbm.at[pl.ds(e0 + G_STRM, G_STRM)], idx1_v[b])
            for cp in gathers(b):
                cp.start()

        def out_copies(k, b):
            e0 = e_base + k * G_BLK
            return [
                pltpu.make_async_copy(row0_v[b],
                                      out_hbm.at[pl.ds(e0, G_STRM)], osem[b]),
                pltpu.make_async_copy(row1_v[b],
                                      out_hbm.at[pl.ds(e0 + G_STRM, G_STRM)],
                                      osem[b]),
            ]

        def do_block(k, b, prefetch):
            for cp in gathers(b):
                cp.wait()
            for cp in out_copies(k, b):
                cp.start()
            for cp in out_copies(k, b):
                cp.wait()
            if prefetch:
                issue_block(k + 2, b)

        issue_block(0, 0)
        issue_block(1, 1)

        def pair_body(o, carry):
            for b in range(2):
                do_block(o * 2 + b, b, True)
            return carry

        lax.fori_loop(0, N_GBLK // 2 - 1, pair_body, 0)
        for b in range(2):
            do_block(N_GBLK - 2 + b, b, False)

    kern = pl.kernel(
        body,
        out_type=jax.ShapeDtypeStruct((MAXB * NA_PAD, H), F32),
        mesh=_sc_mesh(),
        scratch_types=scratch,
    )
    return kern(mb, idx_sm)


def _tc_a2b_combine(nei, base):
    with_base = base is not None
    RB = 2048

    def body(*refs):
        nei_ref = refs[0]
        o_ref = refs[-1]
        v = [nei_ref[j] for j in range(MAXB)]
        s = ((v[0] + v[4]) + v[2]) + ((v[1] + v[5]) + v[3])
        m = jnp.maximum(jnp.maximum(jnp.maximum(v[0], v[4]), v[2]),
                        jnp.maximum(jnp.maximum(v[1], v[5]), v[3]))
        r = s * m
        if with_base:
            r = r + refs[1][...]
        o_ref[...] = r

    in_specs = [pl.BlockSpec((MAXB, RB, H), lambda i: (0, i, 0))]
    args = [nei.reshape(MAXB, NA_PAD, H)]
    if with_base:
        in_specs.append(pl.BlockSpec((RB, H), lambda i: (i, 0)))
        args.append(base)

    return pl.pallas_call(
        body,
        grid=(NA_PAD // RB,),
        in_specs=in_specs,
        out_specs=pl.BlockSpec((RB, H), lambda i: (i, 0)),
        out_shape=jax.ShapeDtypeStruct((NA_PAD, H), F32),
    )(*args)


def _sc_a2b_combine(mb, idx_sm, base):
    nei = _sc_gather6(mb, idx_sm)
    return _tc_a2b_combine(nei, base)


def _sc_bond_diff(ma, mb, b2a_p, b2revb_p):
    scratch = []
    for _ in range(2):
        scratch += [
            pltpu.VMEM((B_BLK,), jnp.int32),
            pltpu.VMEM((B_BLK,), jnp.int32),
            pltpu.VMEM((B_BLK, H), F32),
            pltpu.VMEM((B_BLK, H), F32),
            pltpu.SemaphoreType.DMA,
            pltpu.SemaphoreType.DMA,
        ]

    def body(ma_hbm, mb_hbm, b2a_hbm, b2revb_hbm, out_hbm, *bufs):
        aidx_v = (bufs[0], bufs[6])
        ridx_v = (bufs[1], bufs[7])
        arow_v = (bufs[2], bufs[8])
        rrow_v = (bufs[3], bufs[9])
        gsem = (bufs[4], bufs[10])
        osem = (bufs[5], bufs[11])
        wid = lax.axis_index("s") * NC + lax.axis_index("c")
        e_base = wid * BONDS_W

        def gathers(b):
            return [
                pltpu.make_async_copy(ma_hbm.at[aidx_v[b]], arow_v[b],
                                      gsem[b]),
                pltpu.make_async_copy(mb_hbm.at[ridx_v[b]], rrow_v[b],
                                      gsem[b]),
            ]

        def issue_block(k, b):
            e0 = e_base + k * B_BLK
            pltpu.sync_copy(b2a_hbm.at[pl.ds(e0, B_BLK)], aidx_v[b])
            pltpu.sync_copy(b2revb_hbm.at[pl.ds(e0, B_BLK)], ridx_v[b])
            for cp in gathers(b):
                cp.start()

        def out_copy(k, b):
            e0 = e_base + k * B_BLK
            return pltpu.make_async_copy(
                arow_v[b], out_hbm.at[pl.ds(e0, B_BLK)], osem[b])

        def do_block(k, b, prefetch):
            for cp in gathers(b):
                cp.wait()

            def row_body(r, c2):
                for c in range(H // 16):
                    sl = pl.ds(c * 16, 16)
                    arow_v[b][r, sl] = arow_v[b][r, sl] - rrow_v[b][r, sl]
                return c2

            lax.fori_loop(0, B_BLK, row_body, 0)
            out_copy(k, b).start()
            out_copy(k, b).wait()
            if prefetch:
                issue_block(k + 2, b)

        issue_block(0, 0)
        issue_block(1, 1)

        def pair_body(o, carry):
            for b in range(2):
                do_block(o * 2 + b, b, True)
            return carry

        lax.fori_loop(0, N_BBLK // 2 - 1, pair_body, 0)
        for b in range(2):
            do_block(N_BBLK - 2 + b, b, False)

    kern = pl.kernel(
        body,
        out_type=jax.ShapeDtypeStruct((NB_PAD, H), F32),
        mesh=_sc_mesh(),
        scratch_types=scratch,
    )
    return kern(ma, mb, b2a_p, b2revb_p)


def _rowmm(xs, wTs, n_out_rows, adds=(), bias=None, act=False):
    nx = len(xs)
    nadd = len(adds)
    grid = (pl.cdiv(n_out_rows, ROW_BLK),)

    def body(*refs):
        xrefs = refs[:nx]
        wrefs = refs[nx:2 * nx]
        arefs = refs[2 * nx:2 * nx + nadd]
        pos = 2 * nx + nadd
        b_ref = refs[pos] if bias is not None else None
        o_ref = refs[-1]
        acc = jnp.dot(xrefs[0][...], wrefs[0][...],
                      preferred_element_type=F32)
        for i in range(1, nx):
            acc = acc + jnp.dot(xrefs[i][...], wrefs[i][...],
                                preferred_element_type=F32)
        for a_ref in arefs:
            acc = acc + a_ref[...]
        if b_ref is not None:
            acc = acc + b_ref[...]
        if act:
            acc = jnp.maximum(acc, 0.0)
        o_ref[...] = acc

    in_specs = (
        [pl.BlockSpec((ROW_BLK, x.shape[1]), lambda i: (i, 0)) for x in xs]
        + [pl.BlockSpec(wT.shape, lambda i: (0, 0)) for wT in wTs]
        + [pl.BlockSpec((ROW_BLK, H), lambda i: (i, 0)) for _ in adds]
    )
    args = list(xs) + list(wTs) + list(adds)
    if bias is not None:
        in_specs.append(pl.BlockSpec(bias.shape, lambda i: (0, 0)))
        args.append(bias)

    return pl.pallas_call(
        body,
        grid=grid,
        in_specs=in_specs,
        out_specs=pl.BlockSpec((ROW_BLK, H), lambda i: (i, 0)),
        out_shape=jax.ShapeDtypeStruct((n_out_rows, H), F32),
    )(*args)


def _h0_max(hs):
    MB = 40

    def body(h_ref, o_ref):
        m = h_ref[:, 0, :]
        for t in range(1, MOLSZ):
            m = jnp.maximum(m, h_ref[:, t, :])
        o_ref[...] = m

    return pl.pallas_call(
        body,
        grid=(NMOL // MB,),
        in_specs=[pl.BlockSpec((MB, MOLSZ, H), lambda i: (i, 0, 0))],
        out_specs=pl.BlockSpec((MB, H), lambda i: (i, 0)),
        out_shape=jax.ShapeDtypeStruct((NMOL, H), F32),
    )(hs)


def _gru_bidir(hs_t, h0, gbias, wih_f, whh_f, bih_f, bhh_f,
               wih_b, whh_b, bih_b, bhh_b):
    def body(hsf_ref, hsb_ref, h0_ref, gb_ref,
             wihf_ref, whhf_ref, bihf_ref, bhhf_ref,
             wihb_ref, whhb_ref, bihb_ref, bhhb_ref,
             of_ref, ob_ref, hf, hb):
        t = pl.program_id(0)

        @pl.when(t == 0)
        def _():
            hf[...] = h0_ref[...]
            hb[...] = h0_ref[...]

        def cell(x_ref, h_scr, wih, whh, bih, bhh):
            x = jnp.maximum(x_ref[0, :, :] + gb_ref[...], 0.0)
            h = h_scr[...]
            gi = jnp.dot(x, wih[...], preferred_element_type=F32) + bih[...]
            gh = jnp.dot(h, whh[...], preferred_element_type=F32) + bhh[...]
            sig = lambda v: 1.0 / (1.0 + jnp.exp(-v))
            tnh = lambda v: 1.0 - 2.0 / (jnp.exp(2.0 * v) + 1.0)
            r = sig(gi[:, :H] + gh[:, :H])
            z = sig(gi[:, H:2 * H] + gh[:, H:2 * H])
            n = tnh(gi[:, 2 * H:] + r * gh[:, 2 * H:])
            hn = (1.0 - z) * n + z * h
            h_scr[...] = hn
            return hn

        of_ref[0, :, :] = cell(hsf_ref, hf, wihf_ref, whhf_ref,
                               bihf_ref, bhhf_ref)
        ob_ref[0, :, :] = cell(hsb_ref, hb, wihb_ref, whhb_ref,
                               bihb_ref, bhhb_ref)

    full = lambda shape: pl.BlockSpec(shape, lambda t: tuple(0 for _ in shape))
    in_specs = [
        pl.BlockSpec((1, NMOL, H), lambda t: (t, 0, 0)),
        pl.BlockSpec((1, NMOL, H), lambda t: (MOLSZ - 1 - t, 0, 0)),
        full((NMOL, H)),
        full((1, H)),
        full((H, 3 * H)), full((H, 3 * H)), full((1, 3 * H)), full((1, 3 * H)),
        full((H, 3 * H)), full((H, 3 * H)), full((1, 3 * H)), full((1, 3 * H)),
    ]
    out_specs = [
        pl.BlockSpec((1, NMOL, H), lambda t: (t, 0, 0)),
        pl.BlockSpec((1, NMOL, H), lambda t: (MOLSZ - 1 - t, 0, 0)),
    ]
    return pl.pallas_call(
        body,
        grid=(MOLSZ,),
        in_specs=in_specs,
        out_specs=out_specs,
        out_shape=[jax.ShapeDtypeStruct((MOLSZ, NMOL, H), F32)] * 2,
        scratch_shapes=[pltpu.VMEM((NMOL, H), F32),
                        pltpu.VMEM((NMOL, H), F32)],
    )(hs_t, hs_t, h0, gbias, wih_f, whh_f, bih_f, bhh_f,
      wih_b, whh_b, bih_b, bhh_b)


def kernel(f_atoms, f_bonds, a2b, b2a, b2revb, a_scope,
           W_i_atom, W_i_bond, W_h_0, W_h_1, W_lr, W_o_w, W_o_b, gru_bias,
           W_ih_f, W_hh_f, b_ih_f, b_hh_f, W_ih_b, W_hh_b, b_ih_b, b_hh_b):
    a2b_flat = jnp.pad(a2b.astype(jnp.int32),
                       ((0, NA_PAD - NA), (0, 0))).T.reshape(-1)
    b2a_p = jnp.pad(b2a.astype(jnp.int32), (0, NB_PAD - NB))
    b2revb_p = jnp.pad(b2revb.astype(jnp.int32), (0, NB_PAD - NB))

    WiaT = W_i_atom.T
    WibT = W_i_bond.T
    Wh0T = W_h_0.T
    Wh1T = W_h_1.T
    Wl1T = W_lr[:, :H].T
    Wl2T = W_lr[:, H:2 * H].T
    Wl3T = W_lr[:, 2 * H:].T
    Wo1T = W_o_w[:, :H].T
    Wo2T = W_o_w[:, H:].T
    WihT_f = W_ih_f.T
    WhhT_f = W_hh_f.T
    WihT_b = W_ih_b.T
    WhhT_b = W_hh_b.T
    gb = gru_bias.reshape(1, H)
    bihf = b_ih_f.reshape(1, 3 * H)
    bhhf = b_hh_f.reshape(1, 3 * H)
    bihb = b_ih_b.reshape(1, 3 * H)
    bhhb = b_hh_b.reshape(1, 3 * H)
    wob = W_o_b.reshape(1, H)

    f_atoms_p = jnp.pad(f_atoms, ((0, NA_PAD - NA), (0, 0)))
    ia = _rowmm([f_atoms_p], [WiaT], NA_PAD, act=True)
    ib = _rowmm([f_bonds], [WibT], NB_PAD, act=True)

    ma = _sc_a2b_combine(ib, a2b_flat, ia)
    diff = _sc_bond_diff(ma, ib, b2a_p, b2revb_p)
    mb = _rowmm([diff], [Wh0T], NB_PAD, adds=[ib], act=True)

    ma = _sc_a2b_combine(mb, a2b_flat, ma)
    diff = _sc_bond_diff(ma, mb, b2a_p, b2revb_p)
    mb = _rowmm([diff], [Wh1T], NB_PAD, adds=[ib], act=True)

    agg = _sc_a2b_combine(mb, a2b_flat, None)

    hidden = _rowmm([agg, ma, ia], [Wl1T, Wl2T, Wl3T], NA_PAD)

    hs = hidden[1:NA].reshape(NMOL, MOLSZ, H)
    h0 = _h0_max(hs)
    hs_t = hs.transpose(1, 0, 2)
    fwd_t, bwd_t = _gru_bidir(hs_t, h0, gb, WihT_f, WhhT_f, bihf, bhhf,
                              WihT_b, WhhT_b, bihb, bhhb)
    fwd = fwd_t.transpose(1, 0, 2).reshape(NMOL * MOLSZ, H)
    bwd = bwd_t.transpose(1, 0, 2).reshape(NMOL * MOLSZ, H)

    m0 = jnp.maximum(hidden[0:1] + gb, 0.0)
    A = jnp.concatenate([m0, fwd], axis=0)
    B = jnp.concatenate([m0, bwd], axis=0)
    return _rowmm([A, B], [Wo1T, Wo2T], NA, bias=wob, act=True)


# speedup vs baseline: 2.3781x; 2.3781x over previous
import functools

import jax
import jax.numpy as jnp
from jax import lax
from jax.experimental import pallas as pl
from jax.experimental.pallas import tpu as pltpu
from jax.experimental.pallas import tpu_sc as plsc

F32 = jnp.float32

H = 128
NA = 50001
NB = 200001
NMOL = 1000
MOLSZ = 50
MAXB = 6
NA_PAD = 53248
NB_PAD = 200704

NC = 2
NS = 16
NW = NC * NS

ATOMS_W = NA_PAD // NW
A_BLK = 64
N_ABLK = ATOMS_W // A_BLK
BONDS_W = NB_PAD // NW
B_BLK = 112
N_BBLK = BONDS_W // B_BLK

ROW_BLK = 2048


def _sc_mesh():
    return plsc.VectorSubcoreMesh(core_axis_name="c", subcore_axis_name="s")


def _sc_gather6(mb, idx_sm):
    G_STRM = 104
    G_BLK = 2 * G_STRM
    GW = MAXB * NA_PAD // NW
    N_GBLK = GW // G_BLK

    scratch = []
    for _ in range(2):
        scratch += [
            pltpu.VMEM((G_STRM,), jnp.int32),
            pltpu.VMEM((G_STRM,), jnp.int32),
            pltpu.VMEM((G_STRM, H), F32),
            pltpu.VMEM((G_STRM, H), F32),
            pltpu.SemaphoreType.DMA,
            pltpu.SemaphoreType.DMA,
        ]

    def body(mb_hbm, idx_hbm, out_hbm, *bufs):
        idx0_v = (bufs[0], bufs[6])
        idx1_v = (bufs[1], bufs[7])
        row0_v = (bufs[2], bufs[8])
        row1_v = (bufs[3], bufs[9])
        gsem = (bufs[4], bufs[10])
        osem = (bufs[5], bufs[11])
        wid = lax.axis_index("s") * NC + lax.axis_index("c")
        e_base = wid * GW

        def gathers(b):
            return [
                pltpu.make_async_copy(mb_hbm.at[idx0_v[b]], row0_v[b],
                                      gsem[b]),
                pltpu.make_async_copy(mb_hbm.at[idx1_v[b]], row1_v[b],
                                      gsem[b]),
            ]

        def issue_block(k, b):
            e0 = e_base + k * G_BLK
            pltpu.sync_copy(idx_hbm.at[pl.ds(e0, G_STRM)], idx0_v[b])
            pltpu.sync_copy(idx_hbm.at[pl.ds(e0 + G_STRM, G_STRM)], idx1_v[b])
            for cp in gathers(b):
                cp.start()

        def out_copies(k, b):
            e0 = e_base + k * G_BLK
            return [
                pltpu.make_async_copy(row0_v[b],
                                      out_hbm.at[pl.ds(e0, G_STRM)], osem[b]),
                pltpu.make_async_copy(row1_v[b],
                                      out_hbm.at[pl.ds(e0 + G_STRM, G_STRM)],
                                      osem[b]),
            ]

        def do_block(k, b, prefetch):
            for cp in gathers(b):
                cp.wait()
            for cp in out_copies(k, b):
                cp.start()
            for cp in out_copies(k, b):
                cp.wait()
            if prefetch:
                issue_block(k + 2, b)

        issue_block(0, 0)
        issue_block(1, 1)

        def pair_body(o, carry):
            for b in range(2):
                do_block(o * 2 + b, b, True)
            return carry

        lax.fori_loop(0, N_GBLK // 2 - 1, pair_body, 0)
        for b in range(2):
            do_block(N_GBLK - 2 + b, b, False)

    kern = pl.kernel(
        body,
        out_type=jax.ShapeDtypeStruct((MAXB * NA_PAD, H), F32),
        mesh=_sc_mesh(),
        scratch_types=scratch,
    )
    return kern(mb, idx_sm)


def _tc_a2b_combine(nei, base):
    with_base = base is not None
    RB = 2048

    def body(*refs):
        nei_ref = refs[0]
        o_ref = refs[-1]
        v = [nei_ref[j] for j in range(MAXB)]
        s = ((v[0] + v[4]) + v[2]) + ((v[1] + v[5]) + v[3])
        m = jnp.maximum(jnp.maximum(jnp.maximum(v[0], v[4]), v[2]),
                        jnp.maximum(jnp.maximum(v[1], v[5]), v[3]))
        r = s * m
        if with_base:
            r = r + refs[1][...]
        o_ref[...] = r

    in_specs = [pl.BlockSpec((MAXB, RB, H), lambda i: (0, i, 0))]
    args = [nei.reshape(MAXB, NA_PAD, H)]
    if with_base:
        in_specs.append(pl.BlockSpec((RB, H), lambda i: (i, 0)))
        args.append(base)

    return pl.pallas_call(
        body,
        grid=(NA_PAD // RB,),
        in_specs=in_specs,
        out_specs=pl.BlockSpec((RB, H), lambda i: (i, 0)),
        out_shape=jax.ShapeDtypeStruct((NA_PAD, H), F32),
    )(*args)


def _sc_a2b_combine(mb, idx_sm, base):
    nei = _sc_gather6(mb, idx_sm)
    return _tc_a2b_combine(nei, base)


def _sc_bond_diff(ma, mb, b2a_p, b2revb_p):
    scratch = []
    for _ in range(2):
        scratch += [
            pltpu.VMEM((B_BLK,), jnp.int32),
            pltpu.VMEM((B_BLK,), jnp.int32),
            pltpu.VMEM((B_BLK, H), F32),
            pltpu.VMEM((B_BLK, H), F32),
            pltpu.SemaphoreType.DMA,
            pltpu.SemaphoreType.DMA,
        ]

    def body(ma_hbm, mb_hbm, b2a_hbm, b2revb_hbm, out_hbm, *bufs):
        aidx_v = (bufs[0], bufs[6])
        ridx_v = (bufs[1], bufs[7])
        arow_v = (bufs[2], bufs[8])
        rrow_v = (bufs[3], bufs[9])
        gsem = (bufs[4], bufs[10])
        osem = (bufs[5], bufs[11])
        wid = lax.axis_index("s") * NC + lax.axis_index("c")
        e_base = wid * BONDS_W

        def gathers(b):
            return [
                pltpu.make_async_copy(ma_hbm.at[aidx_v[b]], arow_v[b],
                                      gsem[b]),
                pltpu.make_async_copy(mb_hbm.at[ridx_v[b]], rrow_v[b],
                                      gsem[b]),
            ]

        def issue_block(k, b):
            e0 = e_base + k * B_BLK
            pltpu.sync_copy(b2a_hbm.at[pl.ds(e0, B_BLK)], aidx_v[b])
            pltpu.sync_copy(b2revb_hbm.at[pl.ds(e0, B_BLK)], ridx_v[b])
            for cp in gathers(b):
                cp.start()

        def out_copy(k, b):
            e0 = e_base + k * B_BLK
            return pltpu.make_async_copy(
                arow_v[b], out_hbm.at[pl.ds(e0, B_BLK)], osem[b])

        def do_block(k, b, prefetch):
            for cp in gathers(b):
                cp.wait()

            def row_body(r, c2):
                for c in range(H // 16):
                    sl = pl.ds(c * 16, 16)
                    arow_v[b][r, sl] = arow_v[b][r, sl] - rrow_v[b][r, sl]
                return c2

            lax.fori_loop(0, B_BLK, row_body, 0)
            out_copy(k, b).start()
            out_copy(k, b).wait()
            if prefetch:
                issue_block(k + 2, b)

        issue_block(0, 0)
        issue_block(1, 1)

        def pair_body(o, carry):
            for b in range(2):
                do_block(o * 2 + b, b, True)
            return carry

        lax.fori_loop(0, N_BBLK // 2 - 1, pair_body, 0)
        for b in range(2):
            do_block(N_BBLK - 2 + b, b, False)

    kern = pl.kernel(
        body,
        out_type=jax.ShapeDtypeStruct((NB_PAD, H), F32),
        mesh=_sc_mesh(),
        scratch_types=scratch,
    )
    return kern(ma, mb, b2a_p, b2revb_p)


def _rowmm(xs, wTs, n_out_rows, adds=(), bias=None, act=False):
    nx = len(xs)
    nadd = len(adds)
    grid = (pl.cdiv(n_out_rows, ROW_BLK),)

    def body(*refs):
        xrefs = refs[:nx]
        wrefs = refs[nx:2 * nx]
        arefs = refs[2 * nx:2 * nx + nadd]
        pos = 2 * nx + nadd
        b_ref = refs[pos] if bias is not None else None
        o_ref = refs[-1]
        acc = jnp.dot(xrefs[0][...], wrefs[0][...],
                      preferred_element_type=F32)
        for i in range(1, nx):
            acc = acc + jnp.dot(xrefs[i][...], wrefs[i][...],
                                preferred_element_type=F32)
        for a_ref in arefs:
            acc = acc + a_ref[...]
        if b_ref is not None:
            acc = acc + b_ref[...]
        if act:
            acc = jnp.maximum(acc, 0.0)
        o_ref[...] = acc

    in_specs = (
        [pl.BlockSpec((ROW_BLK, x.shape[1]), lambda i: (i, 0)) for x in xs]
        + [pl.BlockSpec(wT.shape, lambda i: (0, 0)) for wT in wTs]
        + [pl.BlockSpec((ROW_BLK, H), lambda i: (i, 0)) for _ in adds]
    )
    args = list(xs) + list(wTs) + list(adds)
    if bias is not None:
        in_specs.append(pl.BlockSpec(bias.shape, lambda i: (0, 0)))
        args.append(bias)

    return pl.pallas_call(
        body,
        grid=grid,
        in_specs=in_specs,
        out_specs=pl.BlockSpec((ROW_BLK, H), lambda i: (i, 0)),
        out_shape=jax.ShapeDtypeStruct((n_out_rows, H), F32),
    )(*args)


def _h0_max(hs):
    MB = 40

    def body(h_ref, o_ref):
        m = h_ref[:, 0, :]
        for t in range(1, MOLSZ):
            m = jnp.maximum(m, h_ref[:, t, :])
        o_ref[...] = m

    return pl.pallas_call(
        body,
        grid=(NMOL // MB,),
        in_specs=[pl.BlockSpec((MB, MOLSZ, H), lambda i: (i, 0, 0))],
        out_specs=pl.BlockSpec((MB, H), lambda i: (i, 0)),
        out_shape=jax.ShapeDtypeStruct((NMOL, H), F32),
    )(hs)


def _gru_bidir(hs_t, h0, gbias, wih_f, whh_f, bih_f, bhh_f,
               wih_b, whh_b, bih_b, bhh_b):
    def body(hsf_ref, hsb_ref, h0_ref, gb_ref,
             wihf_ref, whhf_ref, bihf_ref, bhhf_ref,
             wihb_ref, whhb_ref, bihb_ref, bhhb_ref,
             of_ref, ob_ref, hf, hb):
        t = pl.program_id(0)

        @pl.when(t == 0)
        def _():
            hf[...] = h0_ref[...]
            hb[...] = h0_ref[...]

        def cell(x_ref, h_scr, wih, whh, bih, bhh):
            x = jnp.maximum(x_ref[0, :, :] + gb_ref[...], 0.0)
            h = h_scr[...]
            gi = jnp.dot(x, wih[...], preferred_element_type=F32) + bih[...]
            gh = jnp.dot(h, whh[...], preferred_element_type=F32) + bhh[...]
            sig = lambda v: 1.0 / (1.0 + jnp.exp(-v))
            tnh = lambda v: 1.0 - 2.0 / (jnp.exp(2.0 * v) + 1.0)
            r = sig(gi[:, :H] + gh[:, :H])
            z = sig(gi[:, H:2 * H] + gh[:, H:2 * H])
            n = tnh(gi[:, 2 * H:] + r * gh[:, 2 * H:])
            hn = (1.0 - z) * n + z * h
            h_scr[...] = hn
            return hn

        of_ref[0, :, :] = cell(hsf_ref, hf, wihf_ref, whhf_ref,
                               bihf_ref, bhhf_ref)
        ob_ref[0, :, :] = cell(hsb_ref, hb, wihb_ref, whhb_ref,
                               bihb_ref, bhhb_ref)

    full = lambda shape: pl.BlockSpec(shape, lambda t: tuple(0 for _ in shape))
    in_specs = [
        pl.BlockSpec((1, NMOL, H), lambda t: (t, 0, 0)),
        pl.BlockSpec((1, NMOL, H), lambda t: (MOLSZ - 1 - t, 0, 0)),
        full((NMOL, H)),
        full((1, H)),
        full((H, 3 * H)), full((H, 3 * H)), full((1, 3 * H)), full((1, 3 * H)),
        full((H, 3 * H)), full((H, 3 * H)), full((1, 3 * H)), full((1, 3 * H)),
    ]
    out_specs = [
        pl.BlockSpec((1, NMOL, H), lambda t: (t, 0, 0)),
        pl.BlockSpec((1, NMOL, H), lambda t: (MOLSZ - 1 - t, 0, 0)),
    ]
    return pl.pallas_call(
        body,
        grid=(MOLSZ,),
        in_specs=in_specs,
        out_specs=out_specs,
        out_shape=[jax.ShapeDtypeStruct((MOLSZ, NMOL, H), F32)] * 2,
        scratch_shapes=[pltpu.VMEM((NMOL, H), F32),
                        pltpu.VMEM((NMOL, H), F32)],
    )(hs_t, hs_t, h0, gbias, wih_f, whh_f, bih_f, bhh_f,
      wih_b, whh_b, bih_b, bhh_b)


def kernel(f_atoms, f_bonds, a2b, b2a, b2revb, a_scope,
           W_i_atom, W_i_bond, W_h_0, W_h_1, W_lr, W_o_w, W_o_b, gru_bias,
           W_ih_f, W_hh_f, b_ih_f, b_hh_f, W_ih_b, W_hh_b, b_ih_b, b_hh_b):
    a2b_pad = (jnp.arange((NA_PAD - NA) * MAXB, dtype=jnp.int32)
               % NB).reshape(NA_PAD - NA, MAXB)
    a2b_flat = jnp.concatenate([a2b.astype(jnp.int32), a2b_pad],
                               axis=0).T.reshape(-1)
    b2a_p = jnp.concatenate(
        [b2a.astype(jnp.int32),
         jnp.arange(NB_PAD - NB, dtype=jnp.int32) % NA])
    b2revb_p = jnp.concatenate(
        [b2revb.astype(jnp.int32),
         jnp.arange(NB_PAD - NB, dtype=jnp.int32) % NB])

    WiaT = W_i_atom.T
    WibT = W_i_bond.T
    Wh0T = W_h_0.T
    Wh1T = W_h_1.T
    Wl1T = W_lr[:, :H].T
    Wl2T = W_lr[:, H:2 * H].T
    Wl3T = W_lr[:, 2 * H:].T
    Wo1T = W_o_w[:, :H].T
    Wo2T = W_o_w[:, H:].T
    WihT_f = W_ih_f.T
    WhhT_f = W_hh_f.T
    WihT_b = W_ih_b.T
    WhhT_b = W_hh_b.T
    gb = gru_bias.reshape(1, H)
    bihf = b_ih_f.reshape(1, 3 * H)
    bhhf = b_hh_f.reshape(1, 3 * H)
    bihb = b_ih_b.reshape(1, 3 * H)
    bhhb = b_hh_b.reshape(1, 3 * H)
    wob = W_o_b.reshape(1, H)

    f_atoms_p = jnp.pad(f_atoms, ((0, NA_PAD - NA), (0, 0)))
    ia = _rowmm([f_atoms_p], [WiaT], NA_PAD, act=True)
    ib = _rowmm([f_bonds], [WibT], NB_PAD, act=True)

    ma = _sc_a2b_combine(ib, a2b_flat, ia)
    diff = _sc_bond_diff(ma, ib, b2a_p, b2revb_p)
    mb = _rowmm([diff], [Wh0T], NB_PAD, adds=[ib], act=True)

    ma = _sc_a2b_combine(mb, a2b_flat, ma)
    diff = _sc_bond_diff(ma, mb, b2a_p, b2revb_p)
    mb = _rowmm([diff], [Wh1T], NB_PAD, adds=[ib], act=True)

    agg = _sc_a2b_combine(mb, a2b_flat, None)

    hidden = _rowmm([agg, ma, ia], [Wl1T, Wl2T, Wl3T], NA_PAD)

    hs = hidden[1:NA].reshape(NMOL, MOLSZ, H)
    h0 = _h0_max(hs)
    hs_t = hs.transpose(1, 0, 2)
    fwd_t, bwd_t = _gru_bidir(hs_t, h0, gb, WihT_f, WhhT_f, bihf, bhhf,
                              WihT_b, WhhT_b, bihb, bhhb)
    fwd = fwd_t.transpose(1, 0, 2).reshape(NMOL * MOLSZ, H)
    bwd = bwd_t.transpose(1, 0, 2).reshape(NMOL * MOLSZ, H)

    m0 = jnp.maximum(hidden[0:1] + gb, 0.0)
    A = jnp.concatenate([m0, fwd], axis=0)
    B = jnp.concatenate([m0, bwd], axis=0)
    return _rowmm([A, B], [Wo1T, Wo2T], NA, bias=wob, act=True)
